# baseline (speedup 1.0000x reference)
import functools

import jax
import jax.numpy as jnp
from jax import lax
from jax.experimental import pallas as pl
from jax.experimental.pallas import tpu as pltpu
from jax.experimental.pallas import tpu_sc as plsc

NTOK = 32768
HIDDEN = 4096
NUM_EXPERTS = 64
TOP_K = 8
BLK = 1024

_NC = 2
_NS = 16
_NW = _NC * _NS
_TPW = NTOK // _NW


def _probs_block(x_ref, w_ref, probs_ref):
    x = x_ref[...]
    w = w_ref[...]
    logits = jax.lax.dot_general(
        x, w, (((1,), (1,)), ((), ())),
        preferred_element_type=jnp.float32,
        precision=jax.lax.Precision.DEFAULT,
    )
    lt = jnp.transpose(logits)
    m = jnp.max(lt, axis=0, keepdims=True)
    e = jnp.exp(lt - m)
    p = e / jnp.sum(e, axis=0, keepdims=True)
    probs_ref[...] = jnp.transpose(p)


def _tc_probs(x, W):
    return pl.pallas_call(
        _probs_block,
        grid=(NTOK // BLK,),
        in_specs=[
            pl.BlockSpec((BLK, HIDDEN), lambda i: (i, 0)),
            pl.BlockSpec((NUM_EXPERTS, HIDDEN), lambda i: (0, 0)),
        ],
        out_specs=pl.BlockSpec((BLK, NUM_EXPERTS), lambda i: (i, 0)),
        out_shape=jax.ShapeDtypeStruct((NTOK, NUM_EXPERTS), jnp.float32),
    )(x, W)


@functools.partial(
    pl.kernel,
    out_type=jax.ShapeDtypeStruct((NTOK, 16), jnp.int32),
    mesh=plsc.VectorSubcoreMesh(core_axis_name="c", subcore_axis_name="s"),
    scratch_types=[
        pltpu.VMEM((_TPW, NUM_EXPERTS), jnp.float32),
        pltpu.VMEM((_TPW, 16), jnp.int32),
    ],
    compiler_params=pltpu.CompilerParams(
        needs_layout_passes=False, use_tc_tiling_on_sc=False),
)
def _sc_topk(probs_hbm, out_hbm, probs_v, idx_v):
    wid = lax.axis_index("s") * _NC + lax.axis_index("c")
    base = wid * _TPW
    pltpu.sync_copy(probs_hbm.at[pl.ds(base, _TPW)], probs_v)

    lane = lax.iota(jnp.int32, 16)

    def body(t, _):
        v = [probs_v[t, pl.ds(16 * j, 16)] for j in range(4)]
        irev = [(63 - (lane + 16 * j)).astype(jnp.float32) for j in range(4)]
        acc = jnp.zeros((16,), jnp.int32)
        for k in range(TOP_K):
            mx = lax.reduce_max(
                jnp.maximum(jnp.maximum(v[0], v[1]),
                            jnp.maximum(v[2], v[3])), axes=(0,))
            sel = lax.reduce_max(
                jnp.maximum(
                    jnp.maximum(jnp.where(v[0] == mx, irev[0], -1.0),
                                jnp.where(v[1] == mx, irev[1], -1.0)),
                    jnp.maximum(jnp.where(v[2] == mx, irev[2], -1.0),
                                jnp.where(v[3] == mx, irev[3], -1.0))),
                axes=(0,))
            amax = 63 - sel.astype(jnp.int32)
            acc = jnp.where(lane == k, amax, acc)
            v = [jnp.where(irev[j] == sel, -1.0, v[j]) for j in range(4)]
        idx_v[t, :] = acc
        return _

    lax.fori_loop(0, _TPW, body, 0)
    pltpu.sync_copy(idx_v, out_hbm.at[pl.ds(base, _TPW)])


def kernel(x, W):
    probs = _tc_probs(x, W)
    idx16 = _sc_topk(probs)
    return (probs, idx16[:, :TOP_K])


# speedup vs baseline: 1.8406x; 1.8406x over previous
import jax
import jax.numpy as jnp
from jax.experimental import pallas as pl
from jax.experimental.pallas import tpu as pltpu

NTOK = 32768
HIDDEN = 4096
NUM_EXPERTS = 64
TOP_K = 8
BLK = 512


def _router_block(x_ref, w_ref, probs_ref, idx_ref):
    x = x_ref[...]
    w = w_ref[...]
    logits = jax.lax.dot_general(
        x, w, (((1,), (1,)), ((), ())),
        preferred_element_type=jnp.float32,
        precision=jax.lax.Precision.DEFAULT,
    )
    lt = jnp.transpose(logits)
    m = jnp.max(lt, axis=0, keepdims=True)
    e = jnp.exp(lt - m)
    p = e / jnp.sum(e, axis=0, keepdims=True)
    probs_ref[...] = jnp.transpose(p)

    iota_e = jax.lax.broadcasted_iota(jnp.int32, (NUM_EXPERTS, BLK), 0)
    irev = (63 - iota_e).astype(jnp.float32)
    work = p
    rows = []
    for _ in range(TOP_K):
        mx = jnp.max(work, axis=0, keepdims=True)
        sel = jnp.max(jnp.where(work == mx, irev, -1.0),
                      axis=0, keepdims=True)
        amax = 63 - sel.astype(jnp.int32)
        rows.append(amax)
        work = jnp.where(iota_e == amax, -1.0, work)
    idx_ref[...] = jnp.transpose(jnp.concatenate(rows, axis=0))


def kernel(x, W):
    grid = (NTOK // BLK,)
    probs, idx = pl.pallas_call(
        _router_block,
        grid=grid,
        in_specs=[
            pl.BlockSpec((BLK, HIDDEN), lambda i: (i, 0)),
            pl.BlockSpec((NUM_EXPERTS, HIDDEN), lambda i: (0, 0)),
        ],
        out_specs=[
            pl.BlockSpec((BLK, NUM_EXPERTS), lambda i: (i, 0)),
            pl.BlockSpec((BLK, TOP_K), lambda i: (i, 0)),
        ],
        out_shape=[
            jax.ShapeDtypeStruct((NTOK, NUM_EXPERTS), jnp.float32),
            jax.ShapeDtypeStruct((NTOK, TOP_K), jnp.int32),
        ],
        compiler_params=pltpu.CompilerParams(
            dimension_semantics=("parallel",),
        ),
    )(x, W)
    return (probs, idx)


# speedup vs baseline: 1.9119x; 1.0388x over previous
import jax
import jax.numpy as jnp
from jax.experimental import pallas as pl
from jax.experimental.pallas import tpu as pltpu

NTOK = 32768
HIDDEN = 4096
NUM_EXPERTS = 64
TOP_K = 8
BLK = 1024
HALF = HIDDEN // 2


def _router_block(xa_ref, xb_ref, w_ref, probs_ref, idx_ref):
    xa = xa_ref[...]
    xb = xb_ref[...]
    w = w_ref[...]
    dims = (((1,), (1,)), ((), ()))
    logits = (
        jax.lax.dot_general(xa, w[:, :HALF], dims,
                            preferred_element_type=jnp.float32,
                            precision=jax.lax.Precision.DEFAULT)
        + jax.lax.dot_general(xb, w[:, HALF:], dims,
                              preferred_element_type=jnp.float32,
                              precision=jax.lax.Precision.DEFAULT)
    )
    lt = jnp.transpose(logits)
    m = jnp.max(lt, axis=0, keepdims=True)
    e = jnp.exp(lt - m)
    p = e / jnp.sum(e, axis=0, keepdims=True)
    probs_ref[...] = jnp.transpose(p)

    iota_e = jax.lax.broadcasted_iota(jnp.int32, (NUM_EXPERTS, BLK), 0)
    irev = (63 - iota_e).astype(jnp.float32)
    work = p
    rows = []
    for _ in range(TOP_K):
        mx = jnp.max(work, axis=0, keepdims=True)
        sel = jnp.max(jnp.where(work == mx, irev, -1.0),
                      axis=0, keepdims=True)
        amax = 63 - sel.astype(jnp.int32)
        rows.append(amax)
        work = jnp.where(iota_e == amax, -1.0, work)
    idx_ref[...] = jnp.transpose(jnp.concatenate(rows, axis=0))


def kernel(x, W):
    grid = (NTOK // BLK,)
    probs, idx = pl.pallas_call(
        _router_block,
        grid=grid,
        in_specs=[
            pl.BlockSpec((BLK, HALF), lambda i: (i, 0)),
            pl.BlockSpec((BLK, HALF), lambda i: (i, 1)),
            pl.BlockSpec((NUM_EXPERTS, HIDDEN), lambda i: (0, 0)),
        ],
        out_specs=[
            pl.BlockSpec((BLK, NUM_EXPERTS), lambda i: (i, 0)),
            pl.BlockSpec((BLK, TOP_K), lambda i: (i, 0)),
        ],
        out_shape=[
            jax.ShapeDtypeStruct((NTOK, NUM_EXPERTS), jnp.float32),
            jax.ShapeDtypeStruct((NTOK, TOP_K), jnp.int32),
        ],
        compiler_params=pltpu.CompilerParams(
            dimension_semantics=("parallel",),
        ),
    )(x, x, W)
    return (probs, idx)
